# baseline (speedup 1.0000x reference)
import functools

import jax
import jax.numpy as jnp
import numpy as np
from jax import lax
from jax.experimental import pallas as pl
from jax.experimental.pallas import tpu as pltpu
from jax.experimental.pallas import tpu_sc as plsc

N, F = 65536, 256
SMOOTHNESS_WEIGHT = 0.01
DERIV_MIN = 0.1
DERIV_MAX = 10.0
DERIV_BOUND_WEIGHT = 1.0

NW = 32
COLS_PER_W = F // NW
SEG = N // 16
KWIN = 512
NWIN = SEG // KWIN
NB = 2048
I32_MIN = np.int32(-2147483648)


def _keys_body(x_ref, k_ref):
    bits = lax.bitcast_convert_type(x_ref[...], jnp.int32)
    m = lax.shift_right_arithmetic(bits, 31)
    keys = bits ^ (m | I32_MIN)
    k_ref[...] = keys.T


def _make_keys(x):
    return pl.pallas_call(
        _keys_body,
        grid=(32,),
        in_specs=[pl.BlockSpec((N // 32, F), lambda i: (i, 0))],
        out_specs=pl.BlockSpec((F, N // 32), lambda i: (0, i)),
        out_shape=jax.ShapeDtypeStruct((F, N), jnp.int32),
    )(x)


def _sc_sort(keys, a, b, c):
    mesh = plsc.VectorSubcoreMesh(core_axis_name="c", subcore_axis_name="s")

    @functools.partial(
        pl.kernel,
        mesh=mesh,
        compiler_params=pltpu.CompilerParams(needs_layout_passes=False),
        out_type=(
            jax.ShapeDtypeStruct((F * 16,), jnp.float32),
            jax.ShapeDtypeStruct((F * 16,), jnp.float32),
            jax.ShapeDtypeStruct((N * F,), jnp.int32),
        ),
        scratch_types=(
            pltpu.VMEM((N + 16,), jnp.int32),
            pltpu.VMEM((16 * KWIN,), jnp.int32),
            pltpu.VMEM((16 * NB,), jnp.int32),
            pltpu.VMEM((F,), jnp.float32),
            pltpu.VMEM((F,), jnp.float32),
            pltpu.VMEM((F,), jnp.float32),
            pltpu.VMEM((16,), jnp.float32),
            pltpu.VMEM((16,), jnp.float32),
            pltpu.SemaphoreType.DMA,
        ),
    )
    def sort_kernel(keys_hbm, a_hbm, b_hbm, c_hbm, sm_hbm, bd_hbm, tmp_hbm,
                    out_v, in_v, hd_v, a_v, b_v, c_v, sm_v, bd_v, dsem):
        cid = lax.axis_index("c")
        sid = lax.axis_index("s")
        wid = cid * 16 + sid

        pltpu.sync_copy(a_hbm, a_v)
        pltpu.sync_copy(b_hbm, b_v)
        pltpu.sync_copy(c_hbm, c_v)

        lane = lax.iota(jnp.int32, 16)
        ctr_off = lane * NB
        win_off = lane * KWIN
        seg_off = lane * SEG
        ones = jnp.ones((16,), jnp.int32)

        def zero_hd(_k, _):
            hd_v[pl.ds(_k * 16, 16)] = jnp.zeros((16,), jnp.int32)
            return 0

        def prefix_in_place(db, carry):
            hs = [hd_v[pl.ds(l * NB + db * 16, 16)] for l in range(16)]
            tot = hs[0]
            for l in range(1, 16):
                tot = tot + hs[l]
            s = plsc.cumsum(tot)
            run = carry + s - tot
            for l in range(16):
                hd_v[pl.ds(l * NB + db * 16, 16)] = run
                run = run + hs[l]
            return carry + jnp.sum(tot, axis=0)

        def digit0(v):
            return v & 2047

        def digit1(v):
            return lax.shift_right_logical(v, 11) & 2047

        def digit2(v):
            return lax.shift_right_logical(v, 22)

        def key_to_x(v):
            bits = jnp.where(v < 0, v ^ I32_MIN, ~v)
            return plsc.bitcast(bits, jnp.float32)

        def deriv(x, av, bv, cv):
            e = jnp.exp((2.0 * cv) * x)
            t = 1.0 - 2.0 / (e + 1.0)
            return av + bv * cv * (1.0 - t * t)

        def stage_window(src_hbm, src0, w):
            copies = [
                pltpu.async_copy(
                    src_hbm.at[pl.ds(src0 + l * SEG + w * KWIN, KWIN)],
                    in_v.at[pl.ds(l * KWIN, KWIN)],
                    dsem)
                for l in range(16)
            ]
            for cp in copies:
                cp.wait()

        def do_column(j, _):
            col = wid * COLS_PER_W + j
            src0 = col * N

            lax.fori_loop(0, 16 * NB // 16, zero_hd, 0, unroll=8)

            def histA_win(w, _):
                stage_window(keys_hbm, src0, w)

                def body(k, _):
                    v = plsc.load_gather(in_v, [win_off + k])
                    plsc.addupdate_scatter(hd_v, [ctr_off + digit0(v)], ones)
                    return 0
                lax.fori_loop(0, KWIN, body, 0)
                return 0
            lax.fori_loop(0, NWIN, histA_win, 0)

            def scatter_pass(src_hbm, dig):
                def pass_win(w, _):
                    stage_window(src_hbm, src0, w)

                    def body(k, _):
                        v = plsc.load_gather(in_v, [win_off + k])
                        cidx = ctr_off + dig(v)
                        pos = plsc.load_gather(hd_v, [cidx])
                        plsc.store_scatter(out_v, [pos], v)
                        plsc.addupdate_scatter(hd_v, [cidx], ones)
                        return 0
                    lax.fori_loop(0, KWIN, body, 0)
                    return 0
                lax.fori_loop(0, NWIN, pass_win, 0)

            def hist_resident(dig):
                lax.fori_loop(0, 16 * NB // 16, zero_hd, 0, unroll=8)

                def body(k, _):
                    v = plsc.load_gather(out_v, [seg_off + k])
                    plsc.addupdate_scatter(hd_v, [ctr_off + dig(v)], ones)
                    return 0
                lax.fori_loop(0, SEG, body, 0)

            lax.fori_loop(0, NB // 16, prefix_in_place, jnp.int32(0))
            scatter_pass(keys_hbm, digit0)
            hist_resident(digit1)
            pltpu.sync_copy(out_v.at[pl.ds(0, N)], tmp_hbm.at[pl.ds(src0, N)])

            lax.fori_loop(0, NB // 16, prefix_in_place, jnp.int32(0))
            scatter_pass(tmp_hbm, digit1)
            hist_resident(digit2)
            pltpu.sync_copy(out_v.at[pl.ds(0, N)], tmp_hbm.at[pl.ds(src0, N)])

            lax.fori_loop(0, NB // 16, prefix_in_place, jnp.int32(0))
            scatter_pass(tmp_hbm, digit2)

            out_v[pl.ds(N, 16)] = plsc.load_gather(
                out_v, [jnp.full((16,), N - 1, jnp.int32)])

            colv = jnp.full((16,), col, jnp.int32)
            av = plsc.load_gather(a_v, [colv])
            bv = plsc.load_gather(b_v, [colv])
            cv = plsc.load_gather(c_v, [colv])

            def pair_terms(base):
                lo = out_v[pl.ds(base, 16)]
                hi = out_v[pl.ds(base + 1, 16)]
                xlo = key_to_x(lo)
                xhi = key_to_x(hi)
                glo = deriv(xlo, av, bv, cv)
                ghi = deriv(xhi, av, bv, cv)
                d2 = (ghi - glo) / (xhi - xlo + 1e-08)
                bm = jnp.maximum(DERIV_MIN - glo, 0.0)
                am = jnp.maximum(glo - DERIV_MAX, 0.0)
                return d2 * d2, bm * bm + am * am

            def sweep(k, acc):
                s0, b0, s1, b1 = acc
                ds0, db0 = pair_terms(k * 16)
                ds1, db1 = pair_terms(N // 2 + k * 16)
                return (s0 + ds0, b0 + db0, s1 + ds1, b1 + db1)

            z = jnp.zeros((16,), jnp.float32)
            s0, b0, s1, b1 = lax.fori_loop(0, N // 32, sweep, (z, z, z, z))
            sm_v[...] = s0 + s1
            bd_v[...] = b0 + b1
            pltpu.sync_copy(sm_v, sm_hbm.at[pl.ds(col * 16, 16)])
            pltpu.sync_copy(bd_v, bd_hbm.at[pl.ds(col * 16, 16)])
            return 0

        lax.fori_loop(0, COLS_PER_W, do_column, 0)

    return sort_kernel(keys, a, b, c)


def _combine_body(s_ref, b_ref, o_ref):
    smooth = jnp.sum(s_ref[...]) / jnp.float32((N - 1) * F)
    bound = jnp.sum(b_ref[...]) / jnp.float32(N * F)
    o_ref[...] = (SMOOTHNESS_WEIGHT * smooth
                  + DERIV_BOUND_WEIGHT * bound) * jnp.ones((1,), jnp.float32)


def kernel(x_samples, a, b, c):
    keys = _make_keys(x_samples)
    keys1d = keys.reshape(N * F)
    sm, bd, _ = _sc_sort(keys1d, a, b, c)
    out = pl.pallas_call(
        _combine_body,
        out_shape=jax.ShapeDtypeStruct((1,), jnp.float32),
    )(sm.reshape(F, 16), bd.reshape(F, 16))
    return out[0]


# speedup vs baseline: 1.7097x; 1.7097x over previous
import functools

import jax
import jax.numpy as jnp
import numpy as np
from jax import lax
from jax.experimental import pallas as pl
from jax.experimental.pallas import tpu as pltpu
from jax.experimental.pallas import tpu_sc as plsc

N, F = 65536, 256
SMOOTHNESS_WEIGHT = 0.01
DERIV_MIN = 0.1
DERIV_MAX = 10.0
DERIV_BOUND_WEIGHT = 1.0

NW = 32
COLS_PER_W = F // NW
NQ = 4
QLEN = N // NQ
CHUNK = 4096
NWIN = QLEN // CHUNK
NB = 2048
I32_MIN = np.int32(-2147483648)


def _keys_body(x_ref, k_ref):
    bits = lax.bitcast_convert_type(x_ref[...], jnp.int32)
    m = lax.shift_right_arithmetic(bits, 31)
    keys = bits ^ (m | I32_MIN)
    k_ref[...] = keys.T


def _make_keys(x):
    return pl.pallas_call(
        _keys_body,
        grid=(32,),
        in_specs=[pl.BlockSpec((N // 32, F), lambda i: (i, 0))],
        out_specs=pl.BlockSpec((F, N // 32), lambda i: (0, i)),
        out_shape=jax.ShapeDtypeStruct((F, N), jnp.int32),
    )(x)


def _sc_sort(keys, a, b, c):
    mesh = plsc.VectorSubcoreMesh(core_axis_name="c", subcore_axis_name="s")

    @functools.partial(
        pl.kernel,
        mesh=mesh,
        compiler_params=pltpu.CompilerParams(needs_layout_passes=False),
        out_type=(
            jax.ShapeDtypeStruct((F * 16,), jnp.float32),
            jax.ShapeDtypeStruct((F * 16,), jnp.float32),
            jax.ShapeDtypeStruct((N * F,), jnp.int32),
        ),
        scratch_types=(
            pltpu.VMEM((N + 16,), jnp.int32),
            pltpu.VMEM((NQ * CHUNK,), jnp.int32),
            pltpu.VMEM((NQ * NB,), jnp.int32),
            pltpu.VMEM((NB,), jnp.int32),
            pltpu.VMEM((NB,), jnp.int32),
            pltpu.VMEM((NB,), jnp.int32),
            pltpu.VMEM((NB,), jnp.int32),
            pltpu.VMEM((F,), jnp.float32),
            pltpu.VMEM((F,), jnp.float32),
            pltpu.VMEM((F,), jnp.float32),
            pltpu.VMEM((16,), jnp.float32),
            pltpu.VMEM((16,), jnp.float32),
        ),
    )
    def sort_kernel(keys_hbm, a_hbm, b_hbm, c_hbm, sm_hbm, bd_hbm, tmp_hbm,
                    out_v, in_v, hist_v, bq0, bq1, bq2, bq3,
                    a_v, b_v, c_v, sm_v, bd_v):
        cid = lax.axis_index("c")
        sid = lax.axis_index("s")
        wid = cid * 16 + sid
        base_refs = (bq0, bq1, bq2, bq3)

        pltpu.sync_copy(a_hbm, a_v)
        pltpu.sync_copy(b_hbm, b_v)
        pltpu.sync_copy(c_hbm, c_v)

        def zero_hist(_k, _):
            hist_v[pl.ds(_k * 16, 16)] = jnp.zeros((16,), jnp.int32)
            return 0

        def prefix_to_bases(_k, carry):
            h0 = hist_v[pl.ds(_k * 16, 16)]
            h1 = hist_v[pl.ds(NB + _k * 16, 16)]
            h2 = hist_v[pl.ds(2 * NB + _k * 16, 16)]
            h3 = hist_v[pl.ds(3 * NB + _k * 16, 16)]
            tot = h0 + h1 + h2 + h3
            s = plsc.cumsum(tot)
            gb = carry + s - tot
            bq0[pl.ds(_k * 16, 16)] = gb
            bq1[pl.ds(_k * 16, 16)] = gb + h0
            bq2[pl.ds(_k * 16, 16)] = gb + h0 + h1
            bq3[pl.ds(_k * 16, 16)] = gb + h0 + h1 + h2
            return carry + jnp.sum(tot, axis=0)

        def hist_add(hd):
            cnt, last = plsc.scan_count(hd)
            plsc.addupdate_scatter(hist_v, [hd], cnt, mask=last)

        def scatter_one(v, d, bref):
            cnt, last = plsc.scan_count(d)
            bse = plsc.load_gather(bref, [d])
            pos = bse + cnt - 1
            plsc.store_scatter(out_v, [pos], v)
            plsc.addupdate_scatter(bref, [d], cnt, mask=last)
            return pos

        def digit0(v):
            return v & 2047

        def digit1(v):
            return lax.shift_right_logical(v, 11) & 2047

        def digit2(v):
            return lax.shift_right_logical(v, 22)

        def key_to_x(v):
            bits = jnp.where(v < 0, v ^ I32_MIN, ~v)
            return plsc.bitcast(bits, jnp.float32)

        def deriv(x, av, bv, cv):
            e = jnp.exp((2.0 * cv) * x)
            t = 1.0 - 2.0 / (e + 1.0)
            return av + bv * cv * (1.0 - t * t)

        def stream_window(src_hbm, src0, w):
            for q in range(NQ):
                pltpu.sync_copy(
                    src_hbm.at[pl.ds(src0 + q * QLEN + w * CHUNK, CHUNK)],
                    in_v.at[pl.ds(q * CHUNK, CHUNK)])

        def do_column(j, _):
            col = wid * COLS_PER_W + j
            src0 = col * N

            lax.fori_loop(0, NQ * NB // 16, zero_hist, 0, unroll=8)

            def histA_win(w, _):
                stream_window(keys_hbm, src0, w)

                def body(k, _):
                    for q in range(NQ):
                        v = in_v[pl.ds(q * CHUNK + k * 16, 16)]
                        hist_add(digit0(v) + (q * NB))
                    return 0
                lax.fori_loop(0, CHUNK // 16, body, 0, unroll=2)
                return 0
            lax.fori_loop(0, NWIN, histA_win, 0)

            def make_scatter_pass(src_hbm, dig, next_dig):
                def pass_win(w, _):
                    stream_window(src_hbm, src0, w)

                    def body(k, _):
                        vs = [in_v[pl.ds(q * CHUNK + k * 16, 16)]
                              for q in range(NQ)]
                        poss = [scatter_one(vs[q], dig(vs[q]), base_refs[q])
                                for q in range(NQ)]
                        if next_dig is not None:
                            for q in range(NQ):
                                qq = lax.shift_right_logical(poss[q], 14)
                                hist_add((qq * NB) + next_dig(vs[q]))
                        return 0
                    lax.fori_loop(0, CHUNK // 16, body, 0, unroll=2)
                    return 0
                return pass_win

            lax.fori_loop(0, NB // 16, prefix_to_bases, jnp.int32(0))
            lax.fori_loop(0, NQ * NB // 16, zero_hist, 0, unroll=8)
            lax.fori_loop(0, NWIN, make_scatter_pass(keys_hbm, digit0, digit1), 0)
            pltpu.sync_copy(out_v.at[pl.ds(0, N)], tmp_hbm.at[pl.ds(src0, N)])

            lax.fori_loop(0, NB // 16, prefix_to_bases, jnp.int32(0))
            lax.fori_loop(0, NQ * NB // 16, zero_hist, 0, unroll=8)
            lax.fori_loop(0, NWIN, make_scatter_pass(tmp_hbm, digit1, digit2), 0)
            pltpu.sync_copy(out_v.at[pl.ds(0, N)], tmp_hbm.at[pl.ds(src0, N)])

            lax.fori_loop(0, NB // 16, prefix_to_bases, jnp.int32(0))
            lax.fori_loop(0, NWIN, make_scatter_pass(tmp_hbm, digit2, None), 0)

            out_v[pl.ds(N, 16)] = plsc.load_gather(
                out_v, [jnp.full((16,), N - 1, jnp.int32)])

            colv = jnp.full((16,), col, jnp.int32)
            av = plsc.load_gather(a_v, [colv])
            bv = plsc.load_gather(b_v, [colv])
            cv = plsc.load_gather(c_v, [colv])

            def pair_terms(base):
                lo = out_v[pl.ds(base, 16)]
                hi = out_v[pl.ds(base + 1, 16)]
                xlo = key_to_x(lo)
                xhi = key_to_x(hi)
                glo = deriv(xlo, av, bv, cv)
                ghi = deriv(xhi, av, bv, cv)
                d2 = (ghi - glo) / (xhi - xlo + 1e-08)
                bm = jnp.maximum(DERIV_MIN - glo, 0.0)
                am = jnp.maximum(glo - DERIV_MAX, 0.0)
                return d2 * d2, bm * bm + am * am

            def sweep(k, acc):
                s0, b0, s1, b1 = acc
                ds0, db0 = pair_terms(k * 16)
                ds1, db1 = pair_terms(N // 2 + k * 16)
                return (s0 + ds0, b0 + db0, s1 + ds1, b1 + db1)

            z = jnp.zeros((16,), jnp.float32)
            s0, b0, s1, b1 = lax.fori_loop(0, N // 32, sweep, (z, z, z, z), unroll=2)
            sm_v[...] = s0 + s1
            bd_v[...] = b0 + b1
            pltpu.sync_copy(sm_v, sm_hbm.at[pl.ds(col * 16, 16)])
            pltpu.sync_copy(bd_v, bd_hbm.at[pl.ds(col * 16, 16)])
            return 0

        lax.fori_loop(0, COLS_PER_W, do_column, 0)

    return sort_kernel(keys, a, b, c)


def _combine_body(s_ref, b_ref, o_ref):
    smooth = jnp.sum(s_ref[...]) / jnp.float32((N - 1) * F)
    bound = jnp.sum(b_ref[...]) / jnp.float32(N * F)
    o_ref[...] = (SMOOTHNESS_WEIGHT * smooth
                  + DERIV_BOUND_WEIGHT * bound) * jnp.ones((1,), jnp.float32)


def kernel(x_samples, a, b, c):
    keys = _make_keys(x_samples)
    keys1d = keys.reshape(N * F)
    sm, bd, _ = _sc_sort(keys1d, a, b, c)
    out = pl.pallas_call(
        _combine_body,
        out_shape=jax.ShapeDtypeStruct((1,), jnp.float32),
    )(sm.reshape(F, 16), bd.reshape(F, 16))
    return out[0]


# speedup vs baseline: 1.8088x; 1.0580x over previous
import functools

import jax
import jax.numpy as jnp
import numpy as np
from jax import lax
from jax.experimental import pallas as pl
from jax.experimental.pallas import tpu as pltpu
from jax.experimental.pallas import tpu_sc as plsc

N, F = 65536, 256
SMOOTHNESS_WEIGHT = 0.01
DERIV_MIN = 0.1
DERIV_MAX = 10.0
DERIV_BOUND_WEIGHT = 1.0

NW = 32
COLS_PER_W = F // NW
NQ = 4
QLEN = N // NQ
CHUNK = 4096
NWIN = QLEN // CHUNK
NB = 2048
I32_MIN = np.int32(-2147483648)


def _keys_body(x_ref, k_ref):
    bits = lax.bitcast_convert_type(x_ref[...], jnp.int32)
    m = lax.shift_right_arithmetic(bits, 31)
    keys = bits ^ (m | I32_MIN)
    k_ref[...] = keys.T


def _make_keys(x):
    return pl.pallas_call(
        _keys_body,
        grid=(32,),
        in_specs=[pl.BlockSpec((N // 32, F), lambda i: (i, 0))],
        out_specs=pl.BlockSpec((F, N // 32), lambda i: (0, i)),
        out_shape=jax.ShapeDtypeStruct((F, N), jnp.int32),
    )(x)


def _sc_sort(keys, a, b, c):
    mesh = plsc.VectorSubcoreMesh(core_axis_name="c", subcore_axis_name="s")

    @functools.partial(
        pl.kernel,
        mesh=mesh,
        compiler_params=pltpu.CompilerParams(needs_layout_passes=False),
        out_type=(
            jax.ShapeDtypeStruct((N * F,), jnp.int32),
        ),
        scratch_types=(
            pltpu.VMEM((N + 16,), jnp.int32),
            pltpu.VMEM((NQ * CHUNK,), jnp.int32),
            pltpu.VMEM((NQ * NB,), jnp.int32),
            pltpu.VMEM((NB,), jnp.int32),
            pltpu.VMEM((NB,), jnp.int32),
            pltpu.VMEM((NB,), jnp.int32),
            pltpu.VMEM((NB,), jnp.int32),
        ),
    )
    def sort_kernel(keys_hbm, tmp_hbm,
                    out_v, in_v, hist_v, bq0, bq1, bq2, bq3):
        cid = lax.axis_index("c")
        sid = lax.axis_index("s")
        wid = cid * 16 + sid
        base_refs = (bq0, bq1, bq2, bq3)

        def zero_hist(_k, _):
            hist_v[pl.ds(_k * 16, 16)] = jnp.zeros((16,), jnp.int32)
            return 0

        def prefix_to_bases(_k, carry):
            h0 = hist_v[pl.ds(_k * 16, 16)]
            h1 = hist_v[pl.ds(NB + _k * 16, 16)]
            h2 = hist_v[pl.ds(2 * NB + _k * 16, 16)]
            h3 = hist_v[pl.ds(3 * NB + _k * 16, 16)]
            tot = h0 + h1 + h2 + h3
            s = plsc.cumsum(tot)
            gb = carry + s - tot
            bq0[pl.ds(_k * 16, 16)] = gb
            bq1[pl.ds(_k * 16, 16)] = gb + h0
            bq2[pl.ds(_k * 16, 16)] = gb + h0 + h1
            bq3[pl.ds(_k * 16, 16)] = gb + h0 + h1 + h2
            return carry + jnp.sum(tot, axis=0)

        def hist_add(hd):
            cnt, last = plsc.scan_count(hd)
            plsc.addupdate_scatter(hist_v, [hd], cnt, mask=last)

        def scatter_one(v, d, bref):
            cnt, last = plsc.scan_count(d)
            bse = plsc.load_gather(bref, [d])
            pos = bse + cnt - 1
            plsc.store_scatter(out_v, [pos], v)
            plsc.addupdate_scatter(bref, [d], cnt, mask=last)
            return pos

        def digit0(v):
            return v & 2047

        def digit1(v):
            return lax.shift_right_logical(v, 11) & 2047

        def digit2(v):
            return lax.shift_right_logical(v, 22)

        def stream_window(src_hbm, src0, w):
            for q in range(NQ):
                pltpu.sync_copy(
                    src_hbm.at[pl.ds(src0 + q * QLEN + w * CHUNK, CHUNK)],
                    in_v.at[pl.ds(q * CHUNK, CHUNK)])

        def do_column(j, _):
            col = wid * COLS_PER_W + j
            src0 = col * N

            lax.fori_loop(0, NQ * NB // 16, zero_hist, 0, unroll=8)

            def histA_win(w, _):
                stream_window(keys_hbm, src0, w)

                def body(k, _):
                    for q in range(NQ):
                        v = in_v[pl.ds(q * CHUNK + k * 16, 16)]
                        hist_add(digit0(v) + (q * NB))
                    return 0
                lax.fori_loop(0, CHUNK // 16, body, 0, unroll=2)
                return 0
            lax.fori_loop(0, NWIN, histA_win, 0)

            def make_scatter_pass(src_hbm, dig, next_dig):
                def pass_win(w, _):
                    stream_window(src_hbm, src0, w)

                    def body(k, _):
                        vs = [in_v[pl.ds(q * CHUNK + k * 16, 16)]
                              for q in range(NQ)]
                        poss = [scatter_one(vs[q], dig(vs[q]), base_refs[q])
                                for q in range(NQ)]
                        if next_dig is not None:
                            for q in range(NQ):
                                qq = lax.shift_right_logical(poss[q], 14)
                                hist_add((qq * NB) + next_dig(vs[q]))
                        return 0
                    lax.fori_loop(0, CHUNK // 16, body, 0, unroll=2)
                    return 0
                return pass_win

            lax.fori_loop(0, NB // 16, prefix_to_bases, jnp.int32(0))
            lax.fori_loop(0, NQ * NB // 16, zero_hist, 0, unroll=8)
            lax.fori_loop(0, NWIN, make_scatter_pass(keys_hbm, digit0, digit1), 0)
            pltpu.sync_copy(out_v.at[pl.ds(0, N)], tmp_hbm.at[pl.ds(src0, N)])

            lax.fori_loop(0, NB // 16, prefix_to_bases, jnp.int32(0))
            lax.fori_loop(0, NQ * NB // 16, zero_hist, 0, unroll=8)
            lax.fori_loop(0, NWIN, make_scatter_pass(tmp_hbm, digit1, digit2), 0)
            pltpu.sync_copy(out_v.at[pl.ds(0, N)], tmp_hbm.at[pl.ds(src0, N)])

            lax.fori_loop(0, NB // 16, prefix_to_bases, jnp.int32(0))
            lax.fori_loop(0, NWIN, make_scatter_pass(tmp_hbm, digit2, None), 0)

            pltpu.sync_copy(out_v.at[pl.ds(0, N)], tmp_hbm.at[pl.ds(src0, N)])
            return 0

        lax.fori_loop(0, COLS_PER_W, do_column, 0)

    return sort_kernel(keys)


def _loss_body(k_ref, a_ref, b_ref, c_ref, o_ref):
    v = k_ref[...]
    bits = jnp.where(v < 0, v ^ I32_MIN, ~v)
    x = lax.bitcast_convert_type(bits, jnp.float32)
    av = a_ref[...].reshape(F // 32, 1)
    bv = b_ref[...].reshape(F // 32, 1)
    cv = c_ref[...].reshape(F // 32, 1)
    t = jnp.tanh(cv * x)
    g = av + bv * cv * (1.0 - t * t)
    x_hi = pltpu.roll(x, N - 1, 1)
    g_hi = pltpu.roll(g, N - 1, 1)
    d2 = (g_hi - g) / (x_hi - x + 1e-08)
    valid = lax.broadcasted_iota(jnp.int32, v.shape, 1) < (N - 1)
    smooth_part = jnp.sum(jnp.where(valid, d2 * d2, 0.0))
    bm = jnp.maximum(DERIV_MIN - g, 0.0)
    am = jnp.maximum(g - DERIV_MAX, 0.0)
    bound_part = jnp.sum(bm * bm + am * am)
    part = (SMOOTHNESS_WEIGHT * smooth_part / jnp.float32((N - 1) * F)
            + DERIV_BOUND_WEIGHT * bound_part / jnp.float32(N * F))

    @pl.when(pl.program_id(0) == 0)
    def _():
        o_ref[...] = jnp.zeros((1,), jnp.float32)

    o_ref[...] += part * jnp.ones((1,), jnp.float32)


def kernel(x_samples, a, b, c):
    keys = _make_keys(x_samples)
    keys1d = keys.reshape(N * F)
    (sorted1d,) = _sc_sort(keys1d, a, b, c)
    skeys = sorted1d.reshape(F, N)
    out = pl.pallas_call(
        _loss_body,
        grid=(32,),
        in_specs=[
            pl.BlockSpec((F // 32, N), lambda i: (i, 0)),
            pl.BlockSpec((1, 1, F // 32), lambda i: (i, 0, 0)),
            pl.BlockSpec((1, 1, F // 32), lambda i: (i, 0, 0)),
            pl.BlockSpec((1, 1, F // 32), lambda i: (i, 0, 0)),
        ],
        out_specs=pl.BlockSpec((1,), lambda i: (0,)),
        out_shape=jax.ShapeDtypeStruct((1,), jnp.float32),
    )(skeys, a.reshape(32, 1, F // 32), b.reshape(32, 1, F // 32), c.reshape(32, 1, F // 32))
    return out[0]


# speedup vs baseline: 2.2308x; 1.2333x over previous
import functools

import jax
import jax.numpy as jnp
import numpy as np
from jax import lax
from jax.experimental import pallas as pl
from jax.experimental.pallas import tpu as pltpu
from jax.experimental.pallas import tpu_sc as plsc

N, F = 65536, 256
SMOOTHNESS_WEIGHT = 0.01
DERIV_MIN = 0.1
DERIV_MAX = 10.0
DERIV_BOUND_WEIGHT = 1.0

NW = 32
COLS_PER_W = F // NW
NQ = 4
QLEN = N // NQ
CHUNK = 4096
NWIN = QLEN // CHUNK
NB = 2048
I32_MIN = np.int32(-2147483648)


def _keys_body(x_ref, k_ref):
    bits = lax.bitcast_convert_type(x_ref[...], jnp.int32)
    m = lax.shift_right_arithmetic(bits, 31)
    keys = bits ^ (m | I32_MIN)
    k_ref[...] = keys.T


def _make_keys(x):
    return pl.pallas_call(
        _keys_body,
        grid=(32,),
        in_specs=[pl.BlockSpec((N // 32, F), lambda i: (i, 0))],
        out_specs=pl.BlockSpec((F, N // 32), lambda i: (0, i)),
        out_shape=jax.ShapeDtypeStruct((F, N), jnp.int32),
    )(x)


def _sc_sort(keys, a, b, c):
    mesh = plsc.VectorSubcoreMesh(core_axis_name="c", subcore_axis_name="s")

    @functools.partial(
        pl.kernel,
        mesh=mesh,
        compiler_params=pltpu.CompilerParams(needs_layout_passes=False),
        out_type=(
            jax.ShapeDtypeStruct((N * F,), jnp.int32),
        ),
        scratch_types=(
            pltpu.VMEM((N + 16,), jnp.int32),
            pltpu.VMEM((NQ * CHUNK,), jnp.int32),
            pltpu.VMEM((NQ * NB,), jnp.int32),
            pltpu.VMEM((NB,), jnp.int32),
            pltpu.VMEM((NB,), jnp.int32),
            pltpu.VMEM((NB,), jnp.int32),
            pltpu.VMEM((NB,), jnp.int32),
        ),
    )
    def sort_kernel(keys_hbm, tmp_hbm,
                    out_v, in_v, hist_v, bq0, bq1, bq2, bq3):
        cid = lax.axis_index("c")
        sid = lax.axis_index("s")
        wid = cid * 16 + sid
        base_refs = (bq0, bq1, bq2, bq3)

        def zero_hist(_k, _):
            hist_v[pl.ds(_k * 16, 16)] = jnp.zeros((16,), jnp.int32)
            return 0

        def prefix_to_bases(_k, carry):
            h0 = hist_v[pl.ds(_k * 16, 16)]
            h1 = hist_v[pl.ds(NB + _k * 16, 16)]
            h2 = hist_v[pl.ds(2 * NB + _k * 16, 16)]
            h3 = hist_v[pl.ds(3 * NB + _k * 16, 16)]
            tot = h0 + h1 + h2 + h3
            s = plsc.cumsum(tot)
            gb = carry + s - tot
            bq0[pl.ds(_k * 16, 16)] = gb
            bq1[pl.ds(_k * 16, 16)] = gb + h0
            bq2[pl.ds(_k * 16, 16)] = gb + h0 + h1
            bq3[pl.ds(_k * 16, 16)] = gb + h0 + h1 + h2
            return carry + jnp.sum(tot, axis=0)

        def hist_add(hd):
            cnt, last = plsc.scan_count(hd)
            plsc.addupdate_scatter(hist_v, [hd], cnt, mask=last)

        def scatter_one(v, d, bref):
            cnt, last = plsc.scan_count(d)
            bse = plsc.load_gather(bref, [d])
            pos = bse + cnt - 1
            plsc.store_scatter(out_v, [pos], v)
            plsc.addupdate_scatter(bref, [d], cnt, mask=last)
            return pos

        def digit0(v):
            return v & 2047

        def digit1(v):
            return lax.shift_right_logical(v, 11) & 2047

        def digit2(v):
            return lax.shift_right_logical(v, 22)

        def stream_window(src_hbm, src0, w):
            for q in range(NQ):
                pltpu.sync_copy(
                    src_hbm.at[pl.ds(src0 + q * QLEN + w * CHUNK, CHUNK)],
                    in_v.at[pl.ds(q * CHUNK, CHUNK)])

        def do_column(j, _):
            col = wid * COLS_PER_W + j
            src0 = col * N

            lax.fori_loop(0, NQ * NB // 16, zero_hist, 0, unroll=8)

            def histA_win(w, _):
                stream_window(keys_hbm, src0, w)

                @plsc.parallel_loop(0, CHUNK // 16, unroll=2)
                def body(k):
                    for q in range(NQ):
                        v = in_v[pl.ds(q * CHUNK + k * 16, 16)]
                        hist_add(digit0(v) + (q * NB))
                return 0
            lax.fori_loop(0, NWIN, histA_win, 0)

            def make_scatter_pass(src_hbm, dig, next_dig):
                def pass_win(w, _):
                    stream_window(src_hbm, src0, w)

                    def body(k, _):
                        vs = [in_v[pl.ds(q * CHUNK + k * 16, 16)]
                              for q in range(NQ)]
                        poss = [scatter_one(vs[q], dig(vs[q]), base_refs[q])
                                for q in range(NQ)]
                        if next_dig is not None:
                            for q in range(NQ):
                                qq = lax.shift_right_logical(poss[q], 14)
                                hist_add((qq * NB) + next_dig(vs[q]))
                        return 0
                    lax.fori_loop(0, CHUNK // 16, body, 0, unroll=4)
                    return 0
                return pass_win

            lax.fori_loop(0, NB // 16, prefix_to_bases, jnp.int32(0))
            lax.fori_loop(0, NQ * NB // 16, zero_hist, 0, unroll=8)
            lax.fori_loop(0, NWIN, make_scatter_pass(keys_hbm, digit0, digit1), 0)
            pltpu.sync_copy(out_v.at[pl.ds(0, N)], tmp_hbm.at[pl.ds(src0, N)])

            lax.fori_loop(0, NB // 16, prefix_to_bases, jnp.int32(0))
            lax.fori_loop(0, NQ * NB // 16, zero_hist, 0, unroll=8)
            lax.fori_loop(0, NWIN, make_scatter_pass(tmp_hbm, digit1, digit2), 0)
            pltpu.sync_copy(out_v.at[pl.ds(0, N)], tmp_hbm.at[pl.ds(src0, N)])

            lax.fori_loop(0, NB // 16, prefix_to_bases, jnp.int32(0))
            lax.fori_loop(0, NWIN, make_scatter_pass(tmp_hbm, digit2, None), 0)

            pltpu.sync_copy(out_v.at[pl.ds(0, N)], tmp_hbm.at[pl.ds(src0, N)])
            return 0

        lax.fori_loop(0, COLS_PER_W, do_column, 0)

    return sort_kernel(keys)


def _loss_body(k_ref, a_ref, b_ref, c_ref, o_ref):
    v = k_ref[...]
    bits = jnp.where(v < 0, v ^ I32_MIN, ~v)
    x = lax.bitcast_convert_type(bits, jnp.float32)
    av = a_ref[...].reshape(F // 32, 1)
    bv = b_ref[...].reshape(F // 32, 1)
    cv = c_ref[...].reshape(F // 32, 1)
    t = jnp.tanh(cv * x)
    g = av + bv * cv * (1.0 - t * t)
    x_hi = pltpu.roll(x, N - 1, 1)
    g_hi = pltpu.roll(g, N - 1, 1)
    d2 = (g_hi - g) / (x_hi - x + 1e-08)
    valid = lax.broadcasted_iota(jnp.int32, v.shape, 1) < (N - 1)
    smooth_part = jnp.sum(jnp.where(valid, d2 * d2, 0.0))
    bm = jnp.maximum(DERIV_MIN - g, 0.0)
    am = jnp.maximum(g - DERIV_MAX, 0.0)
    bound_part = jnp.sum(bm * bm + am * am)
    part = (SMOOTHNESS_WEIGHT * smooth_part / jnp.float32((N - 1) * F)
            + DERIV_BOUND_WEIGHT * bound_part / jnp.float32(N * F))

    @pl.when(pl.program_id(0) == 0)
    def _():
        o_ref[...] = jnp.zeros((1,), jnp.float32)

    o_ref[...] += part * jnp.ones((1,), jnp.float32)


def kernel(x_samples, a, b, c):
    keys = _make_keys(x_samples)
    keys1d = keys.reshape(N * F)
    (sorted1d,) = _sc_sort(keys1d, a, b, c)
    skeys = sorted1d.reshape(F, N)
    out = pl.pallas_call(
        _loss_body,
        grid=(32,),
        in_specs=[
            pl.BlockSpec((F // 32, N), lambda i: (i, 0)),
            pl.BlockSpec((1, 1, F // 32), lambda i: (i, 0, 0)),
            pl.BlockSpec((1, 1, F // 32), lambda i: (i, 0, 0)),
            pl.BlockSpec((1, 1, F // 32), lambda i: (i, 0, 0)),
        ],
        out_specs=pl.BlockSpec((1,), lambda i: (0,)),
        out_shape=jax.ShapeDtypeStruct((1,), jnp.float32),
    )(skeys, a.reshape(32, 1, F // 32), b.reshape(32, 1, F // 32), c.reshape(32, 1, F // 32))
    return out[0]


# speedup vs baseline: 2.2361x; 1.0024x over previous
import functools

import jax
import jax.numpy as jnp
import numpy as np
from jax import lax
from jax.experimental import pallas as pl
from jax.experimental.pallas import tpu as pltpu
from jax.experimental.pallas import tpu_sc as plsc

N, F = 65536, 256
SMOOTHNESS_WEIGHT = 0.01
DERIV_MIN = 0.1
DERIV_MAX = 10.0
DERIV_BOUND_WEIGHT = 1.0

NW = 32
COLS_PER_W = F // NW
NQ = 4
QLEN = N // NQ
CHUNK = 4096
NWIN = QLEN // CHUNK
NB = 2048
I32_MIN = np.int32(-2147483648)


def _keys_body(x_ref, k_ref):
    bits = lax.bitcast_convert_type(x_ref[...], jnp.int32)
    m = lax.shift_right_arithmetic(bits, 31)
    keys = bits ^ (m | I32_MIN)
    k_ref[...] = keys.T


def _make_keys(x):
    return pl.pallas_call(
        _keys_body,
        grid=(32,),
        in_specs=[pl.BlockSpec((N // 32, F), lambda i: (i, 0))],
        out_specs=pl.BlockSpec((F, N // 32), lambda i: (0, i)),
        out_shape=jax.ShapeDtypeStruct((F, N), jnp.int32),
    )(x)


def _sc_sort(keys, a, b, c):
    mesh = plsc.VectorSubcoreMesh(core_axis_name="c", subcore_axis_name="s")

    @functools.partial(
        pl.kernel,
        mesh=mesh,
        compiler_params=pltpu.CompilerParams(needs_layout_passes=False),
        out_type=(
            jax.ShapeDtypeStruct((N * F,), jnp.int32),
        ),
        scratch_types=(
            pltpu.VMEM((N + 16,), jnp.int32),
            pltpu.VMEM((NQ * CHUNK,), jnp.int32),
            pltpu.VMEM((NQ * NB,), jnp.int32),
            pltpu.VMEM((NB,), jnp.int32),
            pltpu.VMEM((NB,), jnp.int32),
            pltpu.VMEM((NB,), jnp.int32),
            pltpu.VMEM((NB,), jnp.int32),
        ),
    )
    def sort_kernel(keys_hbm, tmp_hbm,
                    out_v, in_v, hist_v, bq0, bq1, bq2, bq3):
        cid = lax.axis_index("c")
        sid = lax.axis_index("s")
        wid = cid * 16 + sid
        base_refs = (bq0, bq1, bq2, bq3)

        def zero_hist(_k, _):
            hist_v[pl.ds(_k * 16, 16)] = jnp.zeros((16,), jnp.int32)
            return 0

        def prefix_to_bases(_k, carry):
            h0 = hist_v[pl.ds(_k * 16, 16)]
            h1 = hist_v[pl.ds(NB + _k * 16, 16)]
            h2 = hist_v[pl.ds(2 * NB + _k * 16, 16)]
            h3 = hist_v[pl.ds(3 * NB + _k * 16, 16)]
            tot = h0 + h1 + h2 + h3
            s = plsc.cumsum(tot)
            gb = carry + s - tot
            bq0[pl.ds(_k * 16, 16)] = gb
            bq1[pl.ds(_k * 16, 16)] = gb + h0
            bq2[pl.ds(_k * 16, 16)] = gb + h0 + h1
            bq3[pl.ds(_k * 16, 16)] = gb + h0 + h1 + h2
            return carry + jnp.sum(tot, axis=0)

        def hist_add(hd):
            cnt, last = plsc.scan_count(hd)
            plsc.addupdate_scatter(hist_v, [hd], cnt, mask=last)

        def scatter_one(v, d, bref):
            cnt, last = plsc.scan_count(d)
            bse = plsc.load_gather(bref, [d])
            pos = bse + cnt - 1
            plsc.store_scatter(out_v, [pos], v)
            plsc.addupdate_scatter(bref, [d], cnt, mask=last)
            return pos

        def digit0(v):
            return v & 2047

        def digit1(v):
            return lax.shift_right_logical(v, 11) & 2047

        def digit2(v):
            return lax.shift_right_logical(v, 22)

        def stream_window(src_hbm, src0, w):
            for q in range(NQ):
                pltpu.sync_copy(
                    src_hbm.at[pl.ds(src0 + q * QLEN + w * CHUNK, CHUNK)],
                    in_v.at[pl.ds(q * CHUNK, CHUNK)])

        def do_column(j, _):
            col = wid * COLS_PER_W + j
            src0 = col * N

            lax.fori_loop(0, NQ * NB // 16, zero_hist, 0, unroll=8)

            def histA_win(w, _):
                stream_window(keys_hbm, src0, w)

                @plsc.parallel_loop(0, CHUNK // 16, unroll=4)
                def body(k):
                    for q in range(NQ):
                        v = in_v[pl.ds(q * CHUNK + k * 16, 16)]
                        hist_add(digit0(v) + (q * NB))
                return 0
            lax.fori_loop(0, NWIN, histA_win, 0)

            def make_scatter_pass(src_hbm, dig, next_dig):
                def pass_win(w, _):
                    stream_window(src_hbm, src0, w)

                    def body(k, _):
                        vs = [in_v[pl.ds(q * CHUNK + k * 16, 16)]
                              for q in range(NQ)]
                        poss = [scatter_one(vs[q], dig(vs[q]), base_refs[q])
                                for q in range(NQ)]
                        if next_dig is not None:
                            for q in range(NQ):
                                qq = lax.shift_right_logical(poss[q], 14)
                                hist_add((qq * NB) + next_dig(vs[q]))
                        return 0
                    lax.fori_loop(0, CHUNK // 16, body, 0, unroll=8)
                    return 0
                return pass_win

            lax.fori_loop(0, NB // 16, prefix_to_bases, jnp.int32(0))
            lax.fori_loop(0, NQ * NB // 16, zero_hist, 0, unroll=8)
            lax.fori_loop(0, NWIN, make_scatter_pass(keys_hbm, digit0, digit1), 0)
            pltpu.sync_copy(out_v.at[pl.ds(0, N)], tmp_hbm.at[pl.ds(src0, N)])

            lax.fori_loop(0, NB // 16, prefix_to_bases, jnp.int32(0))
            lax.fori_loop(0, NQ * NB // 16, zero_hist, 0, unroll=8)
            lax.fori_loop(0, NWIN, make_scatter_pass(tmp_hbm, digit1, digit2), 0)
            pltpu.sync_copy(out_v.at[pl.ds(0, N)], tmp_hbm.at[pl.ds(src0, N)])

            lax.fori_loop(0, NB // 16, prefix_to_bases, jnp.int32(0))
            lax.fori_loop(0, NWIN, make_scatter_pass(tmp_hbm, digit2, None), 0)

            pltpu.sync_copy(out_v.at[pl.ds(0, N)], tmp_hbm.at[pl.ds(src0, N)])
            return 0

        lax.fori_loop(0, COLS_PER_W, do_column, 0)

    return sort_kernel(keys)


def _loss_body(k_ref, a_ref, b_ref, c_ref, o_ref):
    v = k_ref[...]
    bits = jnp.where(v < 0, v ^ I32_MIN, ~v)
    x = lax.bitcast_convert_type(bits, jnp.float32)
    av = a_ref[...].reshape(F // 32, 1)
    bv = b_ref[...].reshape(F // 32, 1)
    cv = c_ref[...].reshape(F // 32, 1)
    t = jnp.tanh(cv * x)
    g = av + bv * cv * (1.0 - t * t)
    x_hi = pltpu.roll(x, N - 1, 1)
    g_hi = pltpu.roll(g, N - 1, 1)
    d2 = (g_hi - g) / (x_hi - x + 1e-08)
    valid = lax.broadcasted_iota(jnp.int32, v.shape, 1) < (N - 1)
    smooth_part = jnp.sum(jnp.where(valid, d2 * d2, 0.0))
    bm = jnp.maximum(DERIV_MIN - g, 0.0)
    am = jnp.maximum(g - DERIV_MAX, 0.0)
    bound_part = jnp.sum(bm * bm + am * am)
    part = (SMOOTHNESS_WEIGHT * smooth_part / jnp.float32((N - 1) * F)
            + DERIV_BOUND_WEIGHT * bound_part / jnp.float32(N * F))

    @pl.when(pl.program_id(0) == 0)
    def _():
        o_ref[...] = jnp.zeros((1,), jnp.float32)

    o_ref[...] += part * jnp.ones((1,), jnp.float32)


def kernel(x_samples, a, b, c):
    keys = _make_keys(x_samples)
    keys1d = keys.reshape(N * F)
    (sorted1d,) = _sc_sort(keys1d, a, b, c)
    skeys = sorted1d.reshape(F, N)
    out = pl.pallas_call(
        _loss_body,
        grid=(32,),
        in_specs=[
            pl.BlockSpec((F // 32, N), lambda i: (i, 0)),
            pl.BlockSpec((1, 1, F // 32), lambda i: (i, 0, 0)),
            pl.BlockSpec((1, 1, F // 32), lambda i: (i, 0, 0)),
            pl.BlockSpec((1, 1, F // 32), lambda i: (i, 0, 0)),
        ],
        out_specs=pl.BlockSpec((1,), lambda i: (0,)),
        out_shape=jax.ShapeDtypeStruct((1,), jnp.float32),
    )(skeys, a.reshape(32, 1, F // 32), b.reshape(32, 1, F // 32), c.reshape(32, 1, F // 32))
    return out[0]


# speedup vs baseline: 2.3407x; 1.0468x over previous
import functools

import jax
import jax.numpy as jnp
import numpy as np
from jax import lax
from jax.experimental import pallas as pl
from jax.experimental.pallas import tpu as pltpu
from jax.experimental.pallas import tpu_sc as plsc

N, F = 65536, 256
SMOOTHNESS_WEIGHT = 0.01
DERIV_MIN = 0.1
DERIV_MAX = 10.0
DERIV_BOUND_WEIGHT = 1.0

NW = 32
COLS_PER_W = F // NW
NQ = 4
QLEN = N // NQ
CHUNK = 4096
NWIN = QLEN // CHUNK
NB = 2048
I32_MIN = np.int32(-2147483648)


def _keys_body(x_ref, k_ref):
    bits = lax.bitcast_convert_type(x_ref[...], jnp.int32)
    m = lax.shift_right_arithmetic(bits, 31)
    keys = bits ^ (m | I32_MIN)
    k_ref[...] = keys.T


def _make_keys(x):
    return pl.pallas_call(
        _keys_body,
        grid=(32,),
        in_specs=[pl.BlockSpec((N // 32, F), lambda i: (i, 0))],
        out_specs=pl.BlockSpec((F, N // 32), lambda i: (0, i)),
        out_shape=jax.ShapeDtypeStruct((F, N), jnp.int32),
    )(x)


def _sc_sort(keys, a, b, c):
    mesh = plsc.VectorSubcoreMesh(core_axis_name="c", subcore_axis_name="s")

    @functools.partial(
        pl.kernel,
        mesh=mesh,
        compiler_params=pltpu.CompilerParams(needs_layout_passes=False),
        out_type=(
            jax.ShapeDtypeStruct((N * F,), jnp.int32),
        ),
        scratch_types=(
            pltpu.VMEM((N + 16,), jnp.int32),
            pltpu.VMEM((NQ * CHUNK,), jnp.int32),
            pltpu.VMEM((NQ * NB,), jnp.int32),
            pltpu.VMEM((NB,), jnp.int32),
            pltpu.VMEM((NB,), jnp.int32),
            pltpu.VMEM((NB,), jnp.int32),
            pltpu.VMEM((NB,), jnp.int32),
        ),
    )
    def sort_kernel(keys_hbm, tmp_hbm,
                    out_v, in_v, hist_v, bq0, bq1, bq2, bq3):
        cid = lax.axis_index("c")
        sid = lax.axis_index("s")
        wid = cid * 16 + sid
        base_refs = (bq0, bq1, bq2, bq3)

        def zero_hist(_k, _):
            hist_v[pl.ds(_k * 16, 16)] = jnp.zeros((16,), jnp.int32)
            return 0

        def prefix_to_bases(_k, carry):
            h0 = hist_v[pl.ds(_k * 16, 16)]
            h1 = hist_v[pl.ds(NB + _k * 16, 16)]
            h2 = hist_v[pl.ds(2 * NB + _k * 16, 16)]
            h3 = hist_v[pl.ds(3 * NB + _k * 16, 16)]
            tot = h0 + h1 + h2 + h3
            s = plsc.cumsum(tot)
            gb = carry + s - tot
            bq0[pl.ds(_k * 16, 16)] = gb
            bq1[pl.ds(_k * 16, 16)] = gb + h0
            bq2[pl.ds(_k * 16, 16)] = gb + h0 + h1
            bq3[pl.ds(_k * 16, 16)] = gb + h0 + h1 + h2
            return carry + jnp.sum(tot, axis=0)

        def hist_add(hd):
            cnt, last = plsc.scan_count(hd)
            plsc.addupdate_scatter(hist_v, [hd], cnt, mask=last)

        def scatter_one(v, d, bref):
            cnt, last = plsc.scan_count(d)
            bse = plsc.load_gather(bref, [d])
            pos = bse + cnt - 1
            plsc.store_scatter(out_v, [pos], v)
            plsc.addupdate_scatter(bref, [d], cnt, mask=last)
            return pos

        def digit0(v):
            return v & 2047

        def digit1(v):
            return lax.shift_right_logical(v, 11) & 2047

        def digit2(v):
            return lax.shift_right_logical(v, 22)

        def stream_window(src_hbm, src0, w):
            for q in range(NQ):
                pltpu.sync_copy(
                    src_hbm.at[pl.ds(src0 + q * QLEN + w * CHUNK, CHUNK)],
                    in_v.at[pl.ds(q * CHUNK, CHUNK)])

        def do_column(j, _):
            col = wid * COLS_PER_W + j
            src0 = col * N

            lax.fori_loop(0, NQ * NB // 16, zero_hist, 0, unroll=8)

            def histA_win(w, _):
                stream_window(keys_hbm, src0, w)

                @plsc.parallel_loop(0, CHUNK // 16, unroll=4)
                def body(k):
                    for q in range(NQ):
                        v = in_v[pl.ds(q * CHUNK + k * 16, 16)]
                        hist_add(digit0(v) + (q * NB))
                return 0
            lax.fori_loop(0, NWIN, histA_win, 0)

            def make_scatter_pass(src_hbm, dig):
                def pass_win(w, _):
                    stream_window(src_hbm, src0, w)

                    def body(k, _):
                        vs = [in_v[pl.ds(q * CHUNK + k * 16, 16)]
                              for q in range(NQ)]
                        for q in range(NQ):
                            scatter_one(vs[q], dig(vs[q]), base_refs[q])
                        return 0
                    lax.fori_loop(0, CHUNK // 16, body, 0, unroll=8)
                    return 0
                return pass_win

            def hist_resident(dig):
                lax.fori_loop(0, NQ * NB // 16, zero_hist, 0, unroll=8)

                @plsc.parallel_loop(0, N // 16, unroll=4)
                def _hist(k):
                    v = out_v[pl.ds(k * 16, 16)]
                    qb = lax.shift_right_logical(k, 10) * NB
                    hist_add(qb + dig(v))

            lax.fori_loop(0, NB // 16, prefix_to_bases, jnp.int32(0))
            lax.fori_loop(0, NWIN, make_scatter_pass(keys_hbm, digit0), 0)
            hist_resident(digit1)
            pltpu.sync_copy(out_v.at[pl.ds(0, N)], tmp_hbm.at[pl.ds(src0, N)])

            lax.fori_loop(0, NB // 16, prefix_to_bases, jnp.int32(0))
            lax.fori_loop(0, NWIN, make_scatter_pass(tmp_hbm, digit1), 0)
            hist_resident(digit2)
            pltpu.sync_copy(out_v.at[pl.ds(0, N)], tmp_hbm.at[pl.ds(src0, N)])

            lax.fori_loop(0, NB // 16, prefix_to_bases, jnp.int32(0))
            lax.fori_loop(0, NWIN, make_scatter_pass(tmp_hbm, digit2), 0)

            pltpu.sync_copy(out_v.at[pl.ds(0, N)], tmp_hbm.at[pl.ds(src0, N)])
            return 0

        lax.fori_loop(0, COLS_PER_W, do_column, 0)

    return sort_kernel(keys)


def _loss_body(k_ref, a_ref, b_ref, c_ref, o_ref):
    v = k_ref[...]
    bits = jnp.where(v < 0, v ^ I32_MIN, ~v)
    x = lax.bitcast_convert_type(bits, jnp.float32)
    av = a_ref[...].reshape(F // 32, 1)
    bv = b_ref[...].reshape(F // 32, 1)
    cv = c_ref[...].reshape(F // 32, 1)
    t = jnp.tanh(cv * x)
    g = av + bv * cv * (1.0 - t * t)
    x_hi = pltpu.roll(x, N - 1, 1)
    g_hi = pltpu.roll(g, N - 1, 1)
    d2 = (g_hi - g) / (x_hi - x + 1e-08)
    valid = lax.broadcasted_iota(jnp.int32, v.shape, 1) < (N - 1)
    smooth_part = jnp.sum(jnp.where(valid, d2 * d2, 0.0))
    bm = jnp.maximum(DERIV_MIN - g, 0.0)
    am = jnp.maximum(g - DERIV_MAX, 0.0)
    bound_part = jnp.sum(bm * bm + am * am)
    part = (SMOOTHNESS_WEIGHT * smooth_part / jnp.float32((N - 1) * F)
            + DERIV_BOUND_WEIGHT * bound_part / jnp.float32(N * F))

    @pl.when(pl.program_id(0) == 0)
    def _():
        o_ref[...] = jnp.zeros((1,), jnp.float32)

    o_ref[...] += part * jnp.ones((1,), jnp.float32)


def kernel(x_samples, a, b, c):
    keys = _make_keys(x_samples)
    keys1d = keys.reshape(N * F)
    (sorted1d,) = _sc_sort(keys1d, a, b, c)
    skeys = sorted1d.reshape(F, N)
    out = pl.pallas_call(
        _loss_body,
        grid=(32,),
        in_specs=[
            pl.BlockSpec((F // 32, N), lambda i: (i, 0)),
            pl.BlockSpec((1, 1, F // 32), lambda i: (i, 0, 0)),
            pl.BlockSpec((1, 1, F // 32), lambda i: (i, 0, 0)),
            pl.BlockSpec((1, 1, F // 32), lambda i: (i, 0, 0)),
        ],
        out_specs=pl.BlockSpec((1,), lambda i: (0,)),
        out_shape=jax.ShapeDtypeStruct((1,), jnp.float32),
    )(skeys, a.reshape(32, 1, F // 32), b.reshape(32, 1, F // 32), c.reshape(32, 1, F // 32))
    return out[0]
